# baseline (speedup 1.0000x reference)
import jax
import jax.numpy as jnp
from jax import lax
from jax.experimental import pallas as pl
from jax.experimental.pallas import tpu as pltpu

_INTERPRET = False

_RADIUS = 2
_T_STEPS = 2


def _relu(x):
    return jnp.maximum(x, 0.0)


def _dot(a, b):
    return jnp.dot(a, b, preferred_element_type=jnp.float32)


def _ident(n):
    return (lax.broadcasted_iota(jnp.int32, (n, n), 0)
            == lax.broadcasted_iota(jnp.int32, (n, n), 1)).astype(jnp.float32)


def _mt(x, ident):
    return lax.dot_general(x, ident, (((0,), (0,)), ((), ())),
                           preferred_element_type=jnp.float32)


def _dot32(a, b):
    return jnp.dot(a, b, preferred_element_type=jnp.float32)


def _mt32(x, ident):
    return lax.dot_general(x, ident, (((0,), (0,)), ((), ())),
                           preferred_element_type=jnp.float32)


def _body(al_ref, bl_ref, adlf_ref, bdlf_ref, mask_ref,
          afw_ref, afb_ref, nfwa_ref, nfwb_ref, nfb_ref,
          wih_ref, whh_ref, bih_ref, bhh_ref,
          awt_ref, awb_ref, ab_ref, atw_ref, atb_ref,
          mwih_ref, mwhh_ref, mbih_ref, mbhh_ref,
          mat_ref, mabo_ref, malb_ref, moaw_ref, moab_ref,
          o_af_ref, o_pre_ref, o_act0_ref, o_act1_ref,
          o_at0_ref, o_at1_ref,
          o_mfv0_ref, o_am0_ref, o_am1_ref,
          o_mfu0_ref, o_mfu1_ref, o_mfu2_ref,
          o_maw_ref):
    BB, A, L = al_ref.shape
    M = bl_ref.shape[2]
    N = adlf_ref.shape[2]
    K = N // L
    H = afw_ref.shape[0]
    f32 = jnp.float32

    afw = afw_ref[...]
    afb = afb_ref[...]
    nfwa = nfwa_ref[...]
    nfwb = nfwb_ref[...]
    nfb = nfb_ref[...]

    adlf_all = jnp.squeeze(adlf_ref[...], axis=1)
    attm_all = (adlf_all != L - 1).astype(f32)
    smm_all = jnp.where(adlf_all == L - 1, -9.0, 0.0).astype(f32)
    mask_all = jnp.squeeze(mask_ref[...], axis=1)
    penal_all = jnp.where(mask_all == 0.0, -900000000.0, 0.0).astype(f32)

    iota_a = lax.broadcasted_iota(jnp.int32, (L, N), 0)
    iota_b = lax.broadcasted_iota(jnp.int32, (M, N), 0)
    id_h = _ident(H)
    id_h32 = id_h.astype(f32)

    oh, act, h, nf = [], [], [], []
    for i in range(BB):
        al_t = al_ref[i]
        bl_t = bl_ref[i]
        pre_t = _dot(afw, al_t) + afb
        o_pre_ref[i] = _mt(pre_t, id_h)
        a = _relu(pre_t)
        act.append(a)
        h.append(a)
        oh_i = (adlf_all[i:i + 1] == iota_a).astype(f32)
        ohb_i = (bdlf_ref[i] == iota_b).astype(f32)
        oh.append(oh_i)
        an_t = _dot(al_t, oh_i)
        bn_t = _dot(bl_t, ohb_i)
        nf.append(_relu(_dot(nfwa, an_t) + _dot(nfwb, bn_t) + nfb))

    att_out_refs = (o_at0_ref, o_at1_ref)
    act_out_refs = (o_act0_ref, o_act1_ref)

    for d in range(_RADIUS):
        awt = awt_ref[d]
        awb = awb_ref[d]
        ab = ab_ref[d]
        atw = atw_ref[d]
        atb = atb_ref[d]
        wih = wih_ref[d]
        whh = whh_ref[d]
        bih = bih_ref[d]
        bhh = bhh_ref[d]

        s_act = jnp.concatenate([_dot(awt, act[i]) for i in range(BB)], axis=0)
        if d == 0:
            s_nb = jnp.concatenate([_dot(awb, nf[i]) for i in range(BB)], axis=0)
        else:
            s_nb = jnp.concatenate(
                [_dot(_dot(awb, act[i]), oh[i]) for i in range(BB)], axis=0)

        sc = [_relu(s_act + s_nb[:, k * L:(k + 1) * L] + ab)
              + smm_all[:, k * L:(k + 1) * L] for k in range(K)]
        smax = sc[0]
        for k in range(1, K):
            smax = jnp.maximum(smax, sc[k])
        ex = [jnp.exp(sc[k] - smax) for k in range(K)]
        den = ex[0]
        for k in range(1, K):
            den = den + ex[k]
        aw = [ex[k] / den * attm_all[:, k * L:(k + 1) * L] for k in range(K)]
        aw_sum = aw[0]
        for k in range(1, K):
            aw_sum = aw_sum + aw[k]

        for i in range(BB):
            att_out_refs[d][i] = jnp.concatenate(
                [aw[k][i:i + 1] for k in range(K)], axis=0)

            if d == 0:
                nfc_t = aw[0][i:i + 1] * nf[i][:, :L]
                for k in range(1, K):
                    nfc_t = nfc_t + aw[k][i:i + 1] * nf[i][:, k * L:(k + 1) * L]
            else:
                nfc_t = aw[0][i:i + 1] * _dot(act[i], oh[i][:, :L])
                for k in range(1, K):
                    nfc_t = nfc_t + aw[k][i:i + 1] * _dot(
                        act[i], oh[i][:, k * L:(k + 1) * L])

            ctx_t = _relu(_dot(atw, nfc_t) + atb * aw_sum[i:i + 1])

            gi = _dot(wih, ctx_t) + bih
            gh = _dot(whh, h[i]) + bhh
            r = jax.nn.sigmoid(gi[:H] + gh[:H])
            z = jax.nn.sigmoid(gi[H:2 * H] + gh[H:2 * H])
            n = jnp.tanh(gi[2 * H:] + r * gh[2 * H:])
            h[i] = (1.0 - z) * n + z * h[i]
            act[i] = _relu(h[i])
            act_out_refs[d][i] = _mt(act[i], id_h)

    for i in range(BB):
        o_af_ref[i] = _mt(h[i], id_h)

    ones_col = jnp.ones((L, 1), f32)
    mfu0 = jnp.concatenate(
        [_mt32(_dot32(h[i] * mask_all[i:i + 1], ones_col), id_h32)
         for i in range(BB)], axis=0)
    mf = jnp.concatenate(
        [_mt32(_dot32(act[i] * mask_all[i:i + 1], ones_col), id_h32)
         for i in range(BB)], axis=0)
    o_mfu0_ref[...] = mfu0.reshape(BB, 1, H)
    o_mfv0_ref[...] = mf.reshape(BB, 1, H)
    act_mol = _relu(mf)

    mat = mat_ref[...]
    mabo = mabo_ref[...]
    malb = malb_ref[...]
    moaw = moaw_ref[...]
    moab = moab_ref[...]
    mwih = mwih_ref[...]
    mwhh = mwhh_ref[...]
    mbih = mbih_ref[...]
    mbhh = mbhh_ref[...]

    am_out_refs = (o_am0_ref, o_am1_ref)
    mfu_out_refs = (o_mfu1_ref, o_mfu2_ref)
    for t in range(_T_STEPS):
        s_atom = jnp.concatenate([_dot32(mabo, act[i]) for i in range(BB)],
                                 axis=0)
        s_mpe = jnp.sum(act_mol * mat, axis=1, keepdims=True)
        ms = _relu(s_mpe + s_atom + malb) + penal_all
        e = jnp.exp(ms - jnp.max(ms, axis=1, keepdims=True))
        maw_all = e / jnp.sum(e, axis=1, keepdims=True) * mask_all
        o_maw_ref[:, t] = maw_all

        mctx = _relu(jnp.concatenate(
            [_mt32(_dot32((_dot32(moaw, act[i]) + moab) * maw_all[i:i + 1],
                          ones_col), id_h32) for i in range(BB)],
            axis=0))

        gi = _dot32(mctx, mwih) + mbih
        gh = _dot32(mf, mwhh) + mbhh
        r = jax.nn.sigmoid(gi[:, :H] + gh[:, :H])
        z = jax.nn.sigmoid(gi[:, H:2 * H] + gh[:, H:2 * H])
        n = jnp.tanh(gi[:, 2 * H:] + r * gh[:, 2 * H:])
        mf = (1.0 - z) * n + z * mf
        mfu_out_refs[t][...] = mf.reshape(BB, 1, H)
        act_mol = _relu(mf)
        am_out_refs[t][...] = act_mol.reshape(BB, 1, H)


def kernel(atom_list, bond_list, atom_mask, atom_degree_list, bond_degree_list,
           atom_fc_w, atom_fc_b, neighbor_fc_w, neighbor_fc_b,
           gru_w_ih, gru_w_hh, gru_b_ih, gru_b_hh,
           align_w, align_b, attend_w, attend_b,
           mol_gru_w_ih, mol_gru_w_hh, mol_gru_b_ih, mol_gru_b_hh,
           mol_align_w, mol_align_b, mol_attend_w, mol_attend_b):
    B, L, A = atom_list.shape
    M, BD = bond_list.shape[1], bond_list.shape[2]
    K = atom_degree_list.shape[2]
    N = L * K
    H = atom_fc_w.shape[1]
    R = _RADIUS
    T = _T_STEPS
    f32 = jnp.float32

    adlf = atom_degree_list.astype(jnp.int32).transpose(0, 2, 1).reshape(B, 1, N)
    bdlf = bond_degree_list.astype(jnp.int32).transpose(0, 2, 1).reshape(B, 1, N)

    args = (
        atom_list.transpose(0, 2, 1), bond_list.transpose(0, 2, 1),
        adlf, bdlf, atom_mask.reshape(B, 1, L),
        atom_fc_w.T, atom_fc_b.reshape(H, 1),
        neighbor_fc_w[:A].T, neighbor_fc_w[A:].T, neighbor_fc_b.reshape(H, 1),
        gru_w_ih, gru_w_hh,
        gru_b_ih.reshape(R, 3 * H, 1), gru_b_hh.reshape(R, 3 * H, 1),
        align_w[:, :H, 0].reshape(R, 1, H), align_w[:, H:, 0].reshape(R, 1, H),
        align_b.reshape(R, 1, 1),
        attend_w.transpose(0, 2, 1), attend_b.reshape(R, H, 1),
        mol_gru_w_ih.T, mol_gru_w_hh.T,
        mol_gru_b_ih.reshape(1, 3 * H), mol_gru_b_hh.reshape(1, 3 * H),
        mol_align_w[:H, 0].reshape(1, H), mol_align_w[H:, 0].reshape(1, H),
        mol_align_b.reshape(1, 1),
        mol_attend_w.T, mol_attend_b.reshape(H, 1),
    )

    BB = 8

    def dspec(shape):
        return pl.BlockSpec((BB,) + shape, lambda b: (b,) + (0,) * len(shape))

    def wspec(arr):
        nd = arr.ndim
        return pl.BlockSpec(arr.shape, lambda b, _n=nd: (0,) * _n)

    in_specs = [
        dspec((A, L)), dspec((BD, M)), dspec((1, N)), dspec((1, N)),
        dspec((1, L)),
    ] + [wspec(a) for a in args[5:]]

    out_shape = [
        jax.ShapeDtypeStruct((B, L, H), f32),
        jax.ShapeDtypeStruct((B, L, H), f32),
        jax.ShapeDtypeStruct((B, L, H), f32),
        jax.ShapeDtypeStruct((B, L, H), f32),
        jax.ShapeDtypeStruct((B, K, L), f32),
        jax.ShapeDtypeStruct((B, K, L), f32),
        jax.ShapeDtypeStruct((B, 1, H), f32),
        jax.ShapeDtypeStruct((B, 1, H), f32),
        jax.ShapeDtypeStruct((B, 1, H), f32),
        jax.ShapeDtypeStruct((B, 1, H), f32),
        jax.ShapeDtypeStruct((B, 1, H), f32),
        jax.ShapeDtypeStruct((B, 1, H), f32),
        jax.ShapeDtypeStruct((B, T, L), f32),
    ]
    out_specs = [
        dspec((L, H)), dspec((L, H)), dspec((L, H)), dspec((L, H)),
        dspec((K, L)), dspec((K, L)),
        dspec((1, H)), dspec((1, H)), dspec((1, H)),
        dspec((1, H)), dspec((1, H)), dspec((1, H)),
        dspec((T, L)),
    ]

    outs = pl.pallas_call(
        _body,
        grid=(B // BB,),
        in_specs=in_specs,
        out_specs=out_specs,
        out_shape=out_shape,
        compiler_params=pltpu.CompilerParams(
            dimension_semantics=("arbitrary",),
        ),
        interpret=_INTERPRET,
    )(*args)

    (af, pre, act0, act1, at0, at1,
     mfv0, am0, am1, mfu0, mfu1, mfu2, maw) = outs

    sq = lambda x: x.reshape(B, H)
    maw0 = maw[:, 0, :].reshape(B, L, 1)
    maw1 = maw[:, 1, :].reshape(B, L, 1)
    return (af, pre, act0, act1,
            at0.transpose(0, 2, 1)[..., None], at1.transpose(0, 2, 1)[..., None],
            sq(mfv0), sq(am0), sq(am1),
            sq(mfu0), sq(mfu1), sq(mfu2),
            maw0, maw1, sq(mfu2))


# speedup vs baseline: 1.0308x; 1.0308x over previous
import jax
import jax.numpy as jnp
from jax import lax
from jax.experimental import pallas as pl
from jax.experimental.pallas import tpu as pltpu

_INTERPRET = False

_RADIUS = 2
_T_STEPS = 2


def _relu(x):
    return jnp.maximum(x, 0.0)


def _dot(a, b):
    return jnp.dot(a, b, preferred_element_type=jnp.float32)


def _ident(n):
    return (lax.broadcasted_iota(jnp.int32, (n, n), 0)
            == lax.broadcasted_iota(jnp.int32, (n, n), 1)).astype(jnp.float32)


def _mt(x, ident):
    return lax.dot_general(x, ident, (((0,), (0,)), ((), ())),
                           preferred_element_type=jnp.float32)


def _dot32(a, b):
    return jnp.dot(a, b, preferred_element_type=jnp.float32)


def _mt32(x, ident):
    return lax.dot_general(x, ident, (((0,), (0,)), ((), ())),
                           preferred_element_type=jnp.float32)


def _body(al_ref, bl_ref, adlf_ref, bdlf_ref, mask_ref,
          afw_ref, afb_ref, nfwa_ref, nfwb_ref, nfb_ref,
          wih_ref, whh_ref, bih_ref, bhh_ref,
          awt_ref, awb_ref, ab_ref, atw_ref, atb_ref,
          mwih_ref, mwhh_ref, mbih_ref, mbhh_ref,
          mat_ref, mabo_ref, malb_ref, moaw_ref, moab_ref,
          o_af_ref, o_pre_ref, o_act0_ref, o_act1_ref,
          o_at0_ref, o_at1_ref,
          o_mfv0_ref, o_am0_ref, o_am1_ref,
          o_mfu0_ref, o_mfu1_ref, o_mfu2_ref,
          o_maw_ref):
    BB, A, L = al_ref.shape
    M = bl_ref.shape[2]
    N = adlf_ref.shape[2]
    K = N // L
    H = afw_ref.shape[0]
    f32 = jnp.float32

    afw = afw_ref[...]
    afb = afb_ref[...]
    nfwa = nfwa_ref[...]
    nfwb = nfwb_ref[...]
    nfb = nfb_ref[...]

    adlf_all = jnp.squeeze(adlf_ref[...], axis=1)
    attm_all = (adlf_all != L - 1).astype(f32)
    smm_all = jnp.where(adlf_all == L - 1, -9.0, 0.0).astype(f32)
    mask_all = jnp.squeeze(mask_ref[...], axis=1)
    penal_all = jnp.where(mask_all == 0.0, -900000000.0, 0.0).astype(f32)

    iota_a = lax.broadcasted_iota(jnp.int32, (L, N), 0)
    iota_b = lax.broadcasted_iota(jnp.int32, (M, N), 0)
    id_h = _ident(H)
    id_h32 = id_h.astype(f32)

    oh, act, h, nf = [], [], [], []
    for i in range(BB):
        al_t = al_ref[i]
        bl_t = bl_ref[i]
        pre_t = _dot(afw, al_t) + afb
        o_pre_ref[i] = _mt(pre_t, id_h)
        a = _relu(pre_t)
        act.append(a)
        h.append(a)
        oh_i = (adlf_all[i:i + 1] == iota_a).astype(f32)
        ohb_i = (bdlf_ref[i] == iota_b).astype(f32)
        oh.append(oh_i)
        an_t = _dot(al_t, oh_i)
        bn_t = _dot(bl_t, ohb_i)
        nf.append(_relu(_dot(nfwa, an_t) + _dot(nfwb, bn_t) + nfb))

    att_out_refs = (o_at0_ref, o_at1_ref)
    act_out_refs = (o_act0_ref, o_act1_ref)

    for d in range(_RADIUS):
        awt = awt_ref[d]
        awb = awb_ref[d]
        ab = ab_ref[d]
        atw = atw_ref[d]
        atb = atb_ref[d]
        wih = wih_ref[d]
        whh = whh_ref[d]
        bih = bih_ref[d]
        bhh = bhh_ref[d]

        s_act = jnp.concatenate([_dot(awt, act[i]) for i in range(BB)], axis=0)
        if d == 0:
            s_nb = jnp.concatenate([_dot(awb, nf[i]) for i in range(BB)], axis=0)
        else:
            s_nb = jnp.concatenate(
                [_dot(_dot(awb, act[i]), oh[i]) for i in range(BB)], axis=0)

        sc = [_relu(s_act + s_nb[:, k * L:(k + 1) * L] + ab)
              + smm_all[:, k * L:(k + 1) * L] for k in range(K)]
        smax = sc[0]
        for k in range(1, K):
            smax = jnp.maximum(smax, sc[k])
        ex = [jnp.exp(sc[k] - smax) for k in range(K)]
        den = ex[0]
        for k in range(1, K):
            den = den + ex[k]
        aw = [ex[k] / den * attm_all[:, k * L:(k + 1) * L] for k in range(K)]
        aw_sum = aw[0]
        for k in range(1, K):
            aw_sum = aw_sum + aw[k]

        for i in range(BB):
            att_out_refs[d][i] = jnp.concatenate(
                [aw[k][i:i + 1] for k in range(K)], axis=0)

            if d == 0:
                nfc_t = aw[0][i:i + 1] * nf[i][:, :L]
                for k in range(1, K):
                    nfc_t = nfc_t + aw[k][i:i + 1] * nf[i][:, k * L:(k + 1) * L]
            else:
                nfc_t = aw[0][i:i + 1] * _dot(act[i], oh[i][:, :L])
                for k in range(1, K):
                    nfc_t = nfc_t + aw[k][i:i + 1] * _dot(
                        act[i], oh[i][:, k * L:(k + 1) * L])

            ctx_t = _relu(_dot(atw, nfc_t) + atb * aw_sum[i:i + 1])

            gi = _dot(wih, ctx_t) + bih
            gh = _dot(whh, h[i]) + bhh
            r = jax.nn.sigmoid(gi[:H] + gh[:H])
            z = jax.nn.sigmoid(gi[H:2 * H] + gh[H:2 * H])
            n = jnp.tanh(gi[2 * H:] + r * gh[2 * H:])
            h[i] = (1.0 - z) * n + z * h[i]
            act[i] = _relu(h[i])
            act_out_refs[d][i] = _mt(act[i], id_h)

    for i in range(BB):
        o_af_ref[i] = _mt(h[i], id_h)

    ones_col = jnp.ones((L, 1), f32)
    mfu0 = jnp.concatenate(
        [_mt32(_dot32(h[i] * mask_all[i:i + 1], ones_col), id_h32)
         for i in range(BB)], axis=0)
    mf = jnp.concatenate(
        [_mt32(_dot32(act[i] * mask_all[i:i + 1], ones_col), id_h32)
         for i in range(BB)], axis=0)
    o_mfu0_ref[...] = mfu0.reshape(BB, 1, H)
    o_mfv0_ref[...] = mf.reshape(BB, 1, H)
    act_mol = _relu(mf)

    mat = mat_ref[...]
    mabo = mabo_ref[...]
    malb = malb_ref[...]
    moaw = moaw_ref[...]
    moab = moab_ref[...]
    mwih = mwih_ref[...]
    mwhh = mwhh_ref[...]
    mbih = mbih_ref[...]
    mbhh = mbhh_ref[...]

    am_out_refs = (o_am0_ref, o_am1_ref)
    mfu_out_refs = (o_mfu1_ref, o_mfu2_ref)
    for t in range(_T_STEPS):
        s_atom = jnp.concatenate([_dot32(mabo, act[i]) for i in range(BB)],
                                 axis=0)
        s_mpe = jnp.sum(act_mol * mat, axis=1, keepdims=True)
        ms = _relu(s_mpe + s_atom + malb) + penal_all
        e = jnp.exp(ms - jnp.max(ms, axis=1, keepdims=True))
        maw_all = e / jnp.sum(e, axis=1, keepdims=True) * mask_all
        o_maw_ref[:, t] = maw_all

        mctx = _relu(jnp.concatenate(
            [_mt32(_dot32((_dot32(moaw, act[i]) + moab) * maw_all[i:i + 1],
                          ones_col), id_h32) for i in range(BB)],
            axis=0))

        gi = _dot32(mctx, mwih) + mbih
        gh = _dot32(mf, mwhh) + mbhh
        r = jax.nn.sigmoid(gi[:, :H] + gh[:, :H])
        z = jax.nn.sigmoid(gi[:, H:2 * H] + gh[:, H:2 * H])
        n = jnp.tanh(gi[:, 2 * H:] + r * gh[:, 2 * H:])
        mf = (1.0 - z) * n + z * mf
        mfu_out_refs[t][...] = mf.reshape(BB, 1, H)
        act_mol = _relu(mf)
        am_out_refs[t][...] = act_mol.reshape(BB, 1, H)


def kernel(atom_list, bond_list, atom_mask, atom_degree_list, bond_degree_list,
           atom_fc_w, atom_fc_b, neighbor_fc_w, neighbor_fc_b,
           gru_w_ih, gru_w_hh, gru_b_ih, gru_b_hh,
           align_w, align_b, attend_w, attend_b,
           mol_gru_w_ih, mol_gru_w_hh, mol_gru_b_ih, mol_gru_b_hh,
           mol_align_w, mol_align_b, mol_attend_w, mol_attend_b):
    B, L, A = atom_list.shape
    M, BD = bond_list.shape[1], bond_list.shape[2]
    K = atom_degree_list.shape[2]
    N = L * K
    H = atom_fc_w.shape[1]
    R = _RADIUS
    T = _T_STEPS
    f32 = jnp.float32

    adlf = atom_degree_list.astype(jnp.int32).transpose(0, 2, 1).reshape(B, 1, N)
    bdlf = bond_degree_list.astype(jnp.int32).transpose(0, 2, 1).reshape(B, 1, N)

    args = (
        atom_list.transpose(0, 2, 1), bond_list.transpose(0, 2, 1),
        adlf, bdlf, atom_mask.reshape(B, 1, L),
        atom_fc_w.T, atom_fc_b.reshape(H, 1),
        neighbor_fc_w[:A].T, neighbor_fc_w[A:].T, neighbor_fc_b.reshape(H, 1),
        gru_w_ih, gru_w_hh,
        gru_b_ih.reshape(R, 3 * H, 1), gru_b_hh.reshape(R, 3 * H, 1),
        align_w[:, :H, 0].reshape(R, 1, H), align_w[:, H:, 0].reshape(R, 1, H),
        align_b.reshape(R, 1, 1),
        attend_w.transpose(0, 2, 1), attend_b.reshape(R, H, 1),
        mol_gru_w_ih.T, mol_gru_w_hh.T,
        mol_gru_b_ih.reshape(1, 3 * H), mol_gru_b_hh.reshape(1, 3 * H),
        mol_align_w[:H, 0].reshape(1, H), mol_align_w[H:, 0].reshape(1, H),
        mol_align_b.reshape(1, 1),
        mol_attend_w.T, mol_attend_b.reshape(H, 1),
    )

    BB = 16

    def dspec(shape):
        return pl.BlockSpec((BB,) + shape, lambda b: (b,) + (0,) * len(shape))

    def wspec(arr):
        nd = arr.ndim
        return pl.BlockSpec(arr.shape, lambda b, _n=nd: (0,) * _n)

    in_specs = [
        dspec((A, L)), dspec((BD, M)), dspec((1, N)), dspec((1, N)),
        dspec((1, L)),
    ] + [wspec(a) for a in args[5:]]

    out_shape = [
        jax.ShapeDtypeStruct((B, L, H), f32),
        jax.ShapeDtypeStruct((B, L, H), f32),
        jax.ShapeDtypeStruct((B, L, H), f32),
        jax.ShapeDtypeStruct((B, L, H), f32),
        jax.ShapeDtypeStruct((B, K, L), f32),
        jax.ShapeDtypeStruct((B, K, L), f32),
        jax.ShapeDtypeStruct((B, 1, H), f32),
        jax.ShapeDtypeStruct((B, 1, H), f32),
        jax.ShapeDtypeStruct((B, 1, H), f32),
        jax.ShapeDtypeStruct((B, 1, H), f32),
        jax.ShapeDtypeStruct((B, 1, H), f32),
        jax.ShapeDtypeStruct((B, 1, H), f32),
        jax.ShapeDtypeStruct((B, T, L), f32),
    ]
    out_specs = [
        dspec((L, H)), dspec((L, H)), dspec((L, H)), dspec((L, H)),
        dspec((K, L)), dspec((K, L)),
        dspec((1, H)), dspec((1, H)), dspec((1, H)),
        dspec((1, H)), dspec((1, H)), dspec((1, H)),
        dspec((T, L)),
    ]

    outs = pl.pallas_call(
        _body,
        grid=(B // BB,),
        in_specs=in_specs,
        out_specs=out_specs,
        out_shape=out_shape,
        compiler_params=pltpu.CompilerParams(
            dimension_semantics=("arbitrary",),
        ),
        interpret=_INTERPRET,
    )(*args)

    (af, pre, act0, act1, at0, at1,
     mfv0, am0, am1, mfu0, mfu1, mfu2, maw) = outs

    sq = lambda x: x.reshape(B, H)
    maw0 = maw[:, 0, :].reshape(B, L, 1)
    maw1 = maw[:, 1, :].reshape(B, L, 1)
    return (af, pre, act0, act1,
            at0.transpose(0, 2, 1)[..., None], at1.transpose(0, 2, 1)[..., None],
            sq(mfv0), sq(am0), sq(am1),
            sq(mfu0), sq(mfu1), sq(mfu2),
            maw0, maw1, sq(mfu2))
